# baseline (speedup 1.0000x reference)
import functools

import jax
import jax.numpy as jnp
from jax import lax
from jax.experimental import pallas as pl
from jax.experimental.pallas import tpu as pltpu
from jax.experimental.pallas import tpu_sc as plsc

N = 10000
E = 320000
D = 128

NC = 2
NS = 16
NW = NC * NS

EPW = E // NW
CH = 80
NCH = 125
EPWP = NCH * CH
PAD = EPWP - EPW
NROWS = N
RPT = 624
REM = N - NS * RPT
REM_OFF = NS * RPT


def _sc_aggregate(h, srcp, dstp):
  mesh = plsc.VectorSubcoreMesh(core_axis_name="c", subcore_axis_name="s")

  @functools.partial(
      pl.kernel,
      mesh=mesh,
      out_type=jax.ShapeDtypeStruct((NC, N, D), jnp.float32),
      scratch_types=[
          pltpu.VMEM((EPWP,), jnp.int32),
          pltpu.VMEM((EPWP,), jnp.int32),
          pltpu.VMEM((CH, D), jnp.float32),
          pltpu.VMEM((CH, D), jnp.float32),
          pltpu.VMEM((CH, D), jnp.float32),
          pltpu.VMEM_SHARED((NROWS, D), jnp.float32),
          pltpu.SemaphoreType.DMA,
          pltpu.SemaphoreType.DMA,
          pltpu.SemaphoreType.DMA,
          pltpu.SemaphoreType.DMA,
      ],
  )
  def agg_kernel(h_hbm, src_hbm, dst_hbm, out_hbm, srcall_v, dstall_v,
                 rows_a, rows_b, rows_c, acc_sh,
                 sem_a, sem_b, sem_c, sem_d):
    cid = lax.axis_index("c")
    sid = lax.axis_index("s")
    wid = sid * NC + cid

    ebase = wid * EPWP
    init_cp = pltpu.async_copy(h_hbm.at[pl.ds(sid * RPT, RPT)],
                               acc_sh.at[pl.ds(sid * RPT, RPT)], sem_a)
    pre_cp = pltpu.async_copy(src_hbm.at[pl.ds(ebase, EPWP)], srcall_v, sem_b)
    dpre_cp = pltpu.async_copy(dst_hbm.at[pl.ds(ebase, EPWP)], dstall_v, sem_d)

    @pl.when(sid == 0)
    def _():
      pltpu.async_copy(h_hbm.at[pl.ds(REM_OFF, REM)],
                       acc_sh.at[pl.ds(REM_OFF, REM)], sem_c).wait()

    init_cp.wait()
    pre_cp.wait()
    dpre_cp.wait()
    plsc.subcore_barrier()

    def gather(c, rows, sem):
      pltpu.async_copy(h_hbm.at[srcall_v.at[pl.ds(c * CH, CH)]], rows, sem)

    def scat(c, rows, sem):
      pltpu.make_async_copy(h_hbm.at[pl.ds(0, CH)], rows, sem).wait()
      pltpu.sync_copy(rows, acc_sh.at[dstall_v.at[pl.ds(c * CH, CH)]],
                      add=True)

    gather(0, rows_a, sem_a)
    gather(1, rows_b, sem_b)

    def body(g, carry):
      c = 3 * g
      gather(c + 2, rows_c, sem_c)
      scat(c, rows_a, sem_a)
      gather(c + 3, rows_a, sem_a)
      scat(c + 1, rows_b, sem_b)
      gather(c + 4, rows_b, sem_b)
      scat(c + 2, rows_c, sem_c)
      return carry

    lax.fori_loop(0, NCH // 3, body, 0)
    scat(NCH - 2, rows_a, sem_a)
    scat(NCH - 1, rows_b, sem_b)
    plsc.subcore_barrier()

    pltpu.sync_copy(acc_sh.at[pl.ds(sid * RPT, RPT)],
                    out_hbm.at[cid, pl.ds(sid * RPT, RPT)])

    @pl.when(sid == 0)
    def _():
      pltpu.sync_copy(acc_sh.at[pl.ds(REM_OFF, REM)],
                      out_hbm.at[cid, pl.ds(REM_OFF, REM)])

  return agg_kernel(h, srcp, dstp)


def _tc_layer(x, p, W, b, final):
  BR = 2000

  def body(x_ref, p_ref, w_ref, bias_ref, o_ref):
    rst = p_ref[0] + p_ref[1] - x_ref[...]
    y = jnp.dot(rst, w_ref[...], preferred_element_type=jnp.float32)
    y = jnp.maximum(y + bias_ref[...], 0.0)
    if final:
      m = jnp.max(y, axis=-1, keepdims=True)
      s = jnp.sum(jnp.exp(y - m), axis=-1, keepdims=True)
      y = y - (m + jnp.log(s))
    o_ref[...] = y

  row_spec = pl.BlockSpec((BR, D), lambda i: (i, 0))
  return pl.pallas_call(
      body,
      grid=(N // BR,),
      in_specs=[
          row_spec,
          pl.BlockSpec((NC, BR, D), lambda i: (0, i, 0)),
          pl.BlockSpec((D, D), lambda i: (0, 0)),
          pl.BlockSpec((1, D), lambda i: (0, 0)),
      ],
      out_specs=row_spec,
      out_shape=jax.ShapeDtypeStruct((N, D), jnp.float32),
  )(x, p, W, b)


def kernel(h, edge_index, W1, b1, W2, b2):
  srcp = edge_index[0]
  dstp = edge_index[1]
  b1r = b1.reshape(1, D)
  b2r = b2.reshape(1, D)

  p = _sc_aggregate(h, srcp, dstp)
  h1 = _tc_layer(h, p, W1, b1r, final=False)
  p2 = _sc_aggregate(h1, srcp, dstp)
  return _tc_layer(h1, p2, W2, b2r, final=True)


# speedup vs baseline: 1.0446x; 1.0446x over previous
import functools

import jax
import jax.numpy as jnp
from jax import lax
from jax.experimental import pallas as pl
from jax.experimental.pallas import tpu as pltpu
from jax.experimental.pallas import tpu_sc as plsc

N = 10000
E = 320000
D = 128

NC = 2
NS = 16
NW = NC * NS

EPW = E // NW
CH = 80
NCH = 125
EPWP = NCH * CH
PAD = EPWP - EPW
NROWS = N
RPT = 624
REM = N - NS * RPT
REM_OFF = NS * RPT


def _sc_aggregate(h, ei):
  mesh = plsc.VectorSubcoreMesh(core_axis_name="c", subcore_axis_name="s")

  @functools.partial(
      pl.kernel,
      mesh=mesh,
      out_type=jax.ShapeDtypeStruct((NC, N, D), jnp.float32),
      scratch_types=[
          pltpu.VMEM((EPWP,), jnp.int32),
          pltpu.VMEM((EPWP,), jnp.int32),
          pltpu.VMEM((CH, D), jnp.float32),
          pltpu.VMEM((CH, D), jnp.float32),
          pltpu.VMEM((CH, D), jnp.float32),
          pltpu.VMEM_SHARED((NROWS, D), jnp.float32),
          pltpu.SemaphoreType.DMA,
          pltpu.SemaphoreType.DMA,
          pltpu.SemaphoreType.DMA,
          pltpu.SemaphoreType.DMA,
      ],
  )
  def agg_kernel(h_hbm, ei_hbm, out_hbm, srcall_v, dstall_v,
                 rows_a, rows_b, rows_c, acc_sh,
                 sem_a, sem_b, sem_c, sem_d):
    cid = lax.axis_index("c")
    sid = lax.axis_index("s")
    wid = sid * NC + cid

    ebase = wid * EPWP
    init_cp = pltpu.async_copy(h_hbm.at[pl.ds(sid * RPT, RPT)],
                               acc_sh.at[pl.ds(sid * RPT, RPT)], sem_a)
    pre_cp = pltpu.async_copy(ei_hbm.at[pl.ds(ebase, EPWP)], srcall_v, sem_b)
    dpre_cp = pltpu.async_copy(ei_hbm.at[pl.ds(E + ebase, EPWP)], dstall_v,
                               sem_d)

    @pl.when(sid == 0)
    def _():
      pltpu.async_copy(h_hbm.at[pl.ds(REM_OFF, REM)],
                       acc_sh.at[pl.ds(REM_OFF, REM)], sem_c).wait()

    init_cp.wait()
    pre_cp.wait()
    dpre_cp.wait()
    plsc.subcore_barrier()

    def gather(c, rows, sem):
      pltpu.async_copy(h_hbm.at[srcall_v.at[pl.ds(c * CH, CH)]], rows, sem)

    def scat(c, rows, sem):
      pltpu.make_async_copy(h_hbm.at[pl.ds(0, CH)], rows, sem).wait()
      pltpu.sync_copy(rows, acc_sh.at[dstall_v.at[pl.ds(c * CH, CH)]],
                      add=True)

    gather(0, rows_a, sem_a)
    gather(1, rows_b, sem_b)

    def body(g, carry):
      c = 3 * g
      gather(c + 2, rows_c, sem_c)
      scat(c, rows_a, sem_a)
      gather(c + 3, rows_a, sem_a)
      scat(c + 1, rows_b, sem_b)
      gather(c + 4, rows_b, sem_b)
      scat(c + 2, rows_c, sem_c)
      return carry

    lax.fori_loop(0, NCH // 3, body, 0)
    scat(NCH - 2, rows_a, sem_a)
    scat(NCH - 1, rows_b, sem_b)
    plsc.subcore_barrier()

    pltpu.sync_copy(acc_sh.at[pl.ds(sid * RPT, RPT)],
                    out_hbm.at[cid, pl.ds(sid * RPT, RPT)])

    @pl.when(sid == 0)
    def _():
      pltpu.sync_copy(acc_sh.at[pl.ds(REM_OFF, REM)],
                      out_hbm.at[cid, pl.ds(REM_OFF, REM)])

  return agg_kernel(h, ei)


def _tc_layer(x, p, W, b, final):
  BR = 2000

  def body(x_ref, p_ref, w_ref, bias_ref, o_ref):
    rst = p_ref[0] + p_ref[1] - x_ref[...]
    y = jnp.dot(rst, w_ref[...], preferred_element_type=jnp.float32)
    y = jnp.maximum(y + bias_ref[...], 0.0)
    if final:
      m = jnp.max(y, axis=-1, keepdims=True)
      s = jnp.sum(jnp.exp(y - m), axis=-1, keepdims=True)
      y = y - (m + jnp.log(s))
    o_ref[...] = y

  row_spec = pl.BlockSpec((BR, D), lambda i: (i, 0))
  return pl.pallas_call(
      body,
      grid=(N // BR,),
      in_specs=[
          row_spec,
          pl.BlockSpec((NC, BR, D), lambda i: (0, i, 0)),
          pl.BlockSpec((D, D), lambda i: (0, 0)),
          pl.BlockSpec((1, D), lambda i: (0, 0)),
      ],
      out_specs=row_spec,
      out_shape=jax.ShapeDtypeStruct((N, D), jnp.float32),
  )(x, p, W, b)


def kernel(h, edge_index, W1, b1, W2, b2):
  ei = edge_index.reshape(2 * E)
  b1r = b1.reshape(1, D)
  b2r = b2.reshape(1, D)

  p = _sc_aggregate(h, ei)
  h1 = _tc_layer(h, p, W1, b1r, final=False)
  p2 = _sc_aggregate(h1, ei)
  return _tc_layer(h1, p2, W2, b2r, final=True)


# speedup vs baseline: 1.0502x; 1.0053x over previous
import functools

import jax
import jax.numpy as jnp
from jax import lax
from jax.experimental import pallas as pl
from jax.experimental.pallas import tpu as pltpu
from jax.experimental.pallas import tpu_sc as plsc

N = 10000
E = 320000
D = 128

NC = 2
NS = 16
NW = NC * NS

EPW = E // NW
CH = 80
NCH = 125
EPWP = NCH * CH
NROWS = N
RPT = 624
REM = N - NS * RPT
REM_OFF = NS * RPT


def _sc_aggregate(h, ei):
  mesh = plsc.VectorSubcoreMesh(core_axis_name="c", subcore_axis_name="s")

  @functools.partial(
      pl.kernel,
      mesh=mesh,
      out_type=jax.ShapeDtypeStruct((NC, N, D), jnp.float32),
      scratch_types=[
          pltpu.VMEM((EPWP,), jnp.int32),
          pltpu.VMEM((EPWP,), jnp.int32),
          pltpu.VMEM((CH, D), jnp.float32),
          pltpu.VMEM((CH, D), jnp.float32),
          pltpu.VMEM((CH, D), jnp.float32),
          pltpu.VMEM_SHARED((NROWS, D), jnp.float32),
          pltpu.SemaphoreType.DMA,
          pltpu.SemaphoreType.DMA,
          pltpu.SemaphoreType.DMA,
          pltpu.SemaphoreType.DMA,
          pltpu.SemaphoreType.DMA,
      ],
  )
  def agg_kernel(h_hbm, ei_hbm, out_hbm, srcall_v, dstall_v,
                 rows_a, rows_b, rows_c, acc_sh,
                 sem_a, sem_b, sem_c, sem_d, sem_e):
    cid = lax.axis_index("c")
    sid = lax.axis_index("s")
    wid = sid * NC + cid

    ebase = wid * EPWP
    init_cp = pltpu.async_copy(h_hbm.at[pl.ds(sid * RPT, RPT)],
                               acc_sh.at[pl.ds(sid * RPT, RPT)], sem_e)
    pre_cp = pltpu.async_copy(ei_hbm.at[pl.ds(ebase, EPWP)], srcall_v, sem_b)
    dpre_cp = pltpu.async_copy(ei_hbm.at[pl.ds(E + ebase, EPWP)], dstall_v,
                               sem_d)

    @pl.when(sid == 0)
    def _():
      pltpu.async_copy(h_hbm.at[pl.ds(REM_OFF, REM)],
                       acc_sh.at[pl.ds(REM_OFF, REM)], sem_c).wait()

    def gather(c, rows, sem):
      pltpu.async_copy(h_hbm.at[srcall_v.at[pl.ds(c * CH, CH)]], rows, sem)

    def scat(c, rows, sem):
      pltpu.make_async_copy(h_hbm.at[pl.ds(0, CH)], rows, sem).wait()
      pltpu.sync_copy(rows, acc_sh.at[dstall_v.at[pl.ds(c * CH, CH)]],
                      add=True)

    pre_cp.wait()
    gather(0, rows_a, sem_a)
    gather(1, rows_b, sem_b)
    init_cp.wait()
    dpre_cp.wait()
    plsc.subcore_barrier()

    def body(g, carry):
      c = 3 * g
      gather(c + 2, rows_c, sem_c)
      scat(c, rows_a, sem_a)
      gather(c + 3, rows_a, sem_a)
      scat(c + 1, rows_b, sem_b)
      gather(c + 4, rows_b, sem_b)
      scat(c + 2, rows_c, sem_c)
      return carry

    lax.fori_loop(0, NCH // 3, body, 0)
    scat(NCH - 2, rows_a, sem_a)
    scat(NCH - 1, rows_b, sem_b)
    plsc.subcore_barrier()

    pltpu.sync_copy(acc_sh.at[pl.ds(sid * RPT, RPT)],
                    out_hbm.at[cid, pl.ds(sid * RPT, RPT)])

    @pl.when(sid == 0)
    def _():
      pltpu.sync_copy(acc_sh.at[pl.ds(REM_OFF, REM)],
                      out_hbm.at[cid, pl.ds(REM_OFF, REM)])

  return agg_kernel(h, ei)


def _tc_layer(x, p, W, b, final):
  BR = 2000

  def body(x_ref, p_ref, w_ref, bias_ref, o_ref):
    rst = p_ref[0] + p_ref[1] - x_ref[...]
    y = jnp.dot(rst, w_ref[...], preferred_element_type=jnp.float32)
    y = jnp.maximum(y + bias_ref[...], 0.0)
    if final:
      m = jnp.max(y, axis=-1, keepdims=True)
      s = jnp.sum(jnp.exp(y - m), axis=-1, keepdims=True)
      y = y - (m + jnp.log(s))
    o_ref[...] = y

  row_spec = pl.BlockSpec((BR, D), lambda i: (i, 0))
  return pl.pallas_call(
      body,
      grid=(N // BR,),
      in_specs=[
          row_spec,
          pl.BlockSpec((NC, BR, D), lambda i: (0, i, 0)),
          pl.BlockSpec((D, D), lambda i: (0, 0)),
          pl.BlockSpec((1, D), lambda i: (0, 0)),
      ],
      out_specs=row_spec,
      out_shape=jax.ShapeDtypeStruct((N, D), jnp.float32),
  )(x, p, W, b)


def kernel(h, edge_index, W1, b1, W2, b2):
  ei = edge_index.reshape(2 * E)
  b1r = b1.reshape(1, D)
  b2r = b2.reshape(1, D)

  p = _sc_aggregate(h, ei)
  h1 = _tc_layer(h, p, W1, b1r, final=False)
  p2 = _sc_aggregate(h1, ei)
  return _tc_layer(h1, p2, W2, b2r, final=True)
